# Initial kernel scaffold; baseline (speedup 1.0000x reference)
import jax
import jax.numpy as jnp
from jax.experimental import pallas as pl


def kernel(adj_norm, image_item_embeds, text_item_embeds, image_user_embeds, text_user_embeds, is_test, user_pre, item_pre, user_emb, item_emb, Wu0, bu0, Wi0, bi0, Wu1, bu1, Wi1, bi1):
    raise NotImplementedError("write your pallas kernel here")


# speedup vs baseline: 2.7252x; 2.7252x over previous
import functools

import jax
import jax.numpy as jnp
from jax.experimental import pallas as pl

_CAT_RATE = 0.3
_D = 128
_BLOCK_ROWS = 2000


def _l2n(x):
    n = jnp.sqrt(jnp.sum(x * x, axis=1, keepdims=True))
    return x / jnp.maximum(n, 1e-12)


def _leaky(x):
    return jnp.where(x >= 0, x, 0.5 * x)


def _body(up_ref, ue_ref, uimg_ref, utxt_ref,
          ip_ref, ie_ref, iimg_ref, itxt_ref,
          wu0_ref, bu0_ref, wi0_ref, bi0_ref,
          wu1_ref, bu1_ref, wi1_ref, bi1_ref,
          u_out_ref, i_out_ref):
    dot_t = functools.partial(
        jax.lax.dot_general,
        dimension_numbers=(((1,), (1,)), ((), ())),
        preferred_element_type=jnp.float32,
    )

    u = (up_ref[...] + ue_ref[...]
         + _CAT_RATE * _l2n(uimg_ref[...])
         + _CAT_RATE * _l2n(utxt_ref[...]))
    u = _leaky(dot_t(u, wu0_ref[...]) + bu0_ref[...]) + u
    u = _leaky(dot_t(u, wu1_ref[...]) + bu1_ref[...]) + u
    u_out_ref[...] = u

    i = (ip_ref[...] + ie_ref[...]
         + _CAT_RATE * _l2n(iimg_ref[...])
         + _CAT_RATE * _l2n(itxt_ref[...]))
    i = _leaky(dot_t(i, wi0_ref[...]) + bi0_ref[...]) + i
    i = _leaky(dot_t(i, wi1_ref[...]) + bi1_ref[...]) + i
    i_out_ref[...] = i


@jax.jit
def _run(user_pre, user_emb, image_user_embeds, text_user_embeds,
         item_pre, item_emb, image_item_embeds, text_item_embeds,
         Wu0, bu0, Wi0, bi0, Wu1, bu1, Wi1, bi1):
    n_rows = user_pre.shape[0]
    grid = (pl.cdiv(n_rows, _BLOCK_ROWS),)

    row_spec = pl.BlockSpec((_BLOCK_ROWS, _D), lambda n: (n, 0))
    w_spec = pl.BlockSpec((_D, _D), lambda n: (0, 0))
    b_spec = pl.BlockSpec((1, _D), lambda n: (0, 0))

    out_shape = jax.ShapeDtypeStruct((n_rows, _D), jnp.float32)
    return pl.pallas_call(
        _body,
        grid=grid,
        in_specs=[row_spec] * 8 + [w_spec, b_spec] * 4,
        out_specs=[row_spec, row_spec],
        out_shape=[out_shape, out_shape],
    )(user_pre, user_emb, image_user_embeds, text_user_embeds,
      item_pre, item_emb, image_item_embeds, text_item_embeds,
      Wu0, bu0.reshape(1, _D), Wi0, bi0.reshape(1, _D),
      Wu1, bu1.reshape(1, _D), Wi1, bi1.reshape(1, _D))


def kernel(adj_norm, image_item_embeds, text_item_embeds, image_user_embeds,
           text_user_embeds, is_test, user_pre, item_pre, user_emb, item_emb,
           Wu0, bu0, Wi0, bi0, Wu1, bu1, Wi1, bi1):
    u, i = _run(user_pre, user_emb, image_user_embeds, text_user_embeds,
                item_pre, item_emb, image_item_embeds, text_item_embeds,
                Wu0, bu0, Wi0, bi0, Wu1, bu1, Wi1, bi1)
    return (u, i)


# speedup vs baseline: 2.9165x; 1.0702x over previous
import functools

import jax
import jax.numpy as jnp
from jax.experimental import pallas as pl

_CAT_RATE = 0.3
_D = 128
_BLOCK_ROWS = 4000


def _l2n(x):
    n = jnp.sqrt(jnp.sum(x * x, axis=1, keepdims=True))
    return x / jnp.maximum(n, 1e-12)


def _leaky(x):
    return jnp.where(x >= 0, x, 0.5 * x)


def _body(up_ref, ue_ref, uimg_ref, utxt_ref,
          ip_ref, ie_ref, iimg_ref, itxt_ref,
          wu0_ref, bu0_ref, wi0_ref, bi0_ref,
          wu1_ref, bu1_ref, wi1_ref, bi1_ref,
          u_out_ref, i_out_ref):
    dot_t = functools.partial(
        jax.lax.dot_general,
        dimension_numbers=(((1,), (1,)), ((), ())),
        preferred_element_type=jnp.float32,
    )

    u = (up_ref[...] + ue_ref[...]
         + _CAT_RATE * _l2n(uimg_ref[...])
         + _CAT_RATE * _l2n(utxt_ref[...]))
    u = _leaky(dot_t(u, wu0_ref[...]) + bu0_ref[...]) + u
    u = _leaky(dot_t(u, wu1_ref[...]) + bu1_ref[...]) + u
    u_out_ref[...] = u

    i = (ip_ref[...] + ie_ref[...]
         + _CAT_RATE * _l2n(iimg_ref[...])
         + _CAT_RATE * _l2n(itxt_ref[...]))
    i = _leaky(dot_t(i, wi0_ref[...]) + bi0_ref[...]) + i
    i = _leaky(dot_t(i, wi1_ref[...]) + bi1_ref[...]) + i
    i_out_ref[...] = i


@jax.jit
def _run(user_pre, user_emb, image_user_embeds, text_user_embeds,
         item_pre, item_emb, image_item_embeds, text_item_embeds,
         Wu0, bu0, Wi0, bi0, Wu1, bu1, Wi1, bi1):
    n_rows = user_pre.shape[0]
    grid = (pl.cdiv(n_rows, _BLOCK_ROWS),)

    row_spec = pl.BlockSpec((_BLOCK_ROWS, _D), lambda n: (n, 0))
    w_spec = pl.BlockSpec((_D, _D), lambda n: (0, 0))
    b_spec = pl.BlockSpec((1, _D), lambda n: (0, 0))

    out_shape = jax.ShapeDtypeStruct((n_rows, _D), jnp.float32)
    return pl.pallas_call(
        _body,
        grid=grid,
        in_specs=[row_spec] * 8 + [w_spec, b_spec] * 4,
        out_specs=[row_spec, row_spec],
        out_shape=[out_shape, out_shape],
    )(user_pre, user_emb, image_user_embeds, text_user_embeds,
      item_pre, item_emb, image_item_embeds, text_item_embeds,
      Wu0, bu0.reshape(1, _D), Wi0, bi0.reshape(1, _D),
      Wu1, bu1.reshape(1, _D), Wi1, bi1.reshape(1, _D))


def kernel(adj_norm, image_item_embeds, text_item_embeds, image_user_embeds,
           text_user_embeds, is_test, user_pre, item_pre, user_emb, item_emb,
           Wu0, bu0, Wi0, bi0, Wu1, bu1, Wi1, bi1):
    u, i = _run(user_pre, user_emb, image_user_embeds, text_user_embeds,
                item_pre, item_emb, image_item_embeds, text_item_embeds,
                Wu0, bu0, Wi0, bi0, Wu1, bu1, Wi1, bi1)
    return (u, i)


# speedup vs baseline: 2.9271x; 1.0036x over previous
import functools

import jax
import jax.numpy as jnp
from jax.experimental import pallas as pl

_CAT_RATE = 0.3
_D = 128
_BLOCK_ROWS = 5000


def _l2n(x):
    n = jnp.sqrt(jnp.sum(x * x, axis=1, keepdims=True))
    return x / jnp.maximum(n, 1e-12)


def _leaky(x):
    return jnp.where(x >= 0, x, 0.5 * x)


def _body(up_ref, ue_ref, uimg_ref, utxt_ref,
          ip_ref, ie_ref, iimg_ref, itxt_ref,
          wu0_ref, bu0_ref, wi0_ref, bi0_ref,
          wu1_ref, bu1_ref, wi1_ref, bi1_ref,
          u_out_ref, i_out_ref):
    dot_t = functools.partial(
        jax.lax.dot_general,
        dimension_numbers=(((1,), (1,)), ((), ())),
        preferred_element_type=jnp.float32,
    )

    u = (up_ref[...] + ue_ref[...]
         + _CAT_RATE * _l2n(uimg_ref[...])
         + _CAT_RATE * _l2n(utxt_ref[...]))
    u = _leaky(dot_t(u, wu0_ref[...]) + bu0_ref[...]) + u
    u = _leaky(dot_t(u, wu1_ref[...]) + bu1_ref[...]) + u
    u_out_ref[...] = u

    i = (ip_ref[...] + ie_ref[...]
         + _CAT_RATE * _l2n(iimg_ref[...])
         + _CAT_RATE * _l2n(itxt_ref[...]))
    i = _leaky(dot_t(i, wi0_ref[...]) + bi0_ref[...]) + i
    i = _leaky(dot_t(i, wi1_ref[...]) + bi1_ref[...]) + i
    i_out_ref[...] = i


@jax.jit
def _run(user_pre, user_emb, image_user_embeds, text_user_embeds,
         item_pre, item_emb, image_item_embeds, text_item_embeds,
         Wu0, bu0, Wi0, bi0, Wu1, bu1, Wi1, bi1):
    n_rows = user_pre.shape[0]
    grid = (pl.cdiv(n_rows, _BLOCK_ROWS),)

    row_spec = pl.BlockSpec((_BLOCK_ROWS, _D), lambda n: (n, 0))
    w_spec = pl.BlockSpec((_D, _D), lambda n: (0, 0))
    b_spec = pl.BlockSpec((1, _D), lambda n: (0, 0))

    out_shape = jax.ShapeDtypeStruct((n_rows, _D), jnp.float32)
    return pl.pallas_call(
        _body,
        grid=grid,
        in_specs=[row_spec] * 8 + [w_spec, b_spec] * 4,
        out_specs=[row_spec, row_spec],
        out_shape=[out_shape, out_shape],
    )(user_pre, user_emb, image_user_embeds, text_user_embeds,
      item_pre, item_emb, image_item_embeds, text_item_embeds,
      Wu0, bu0.reshape(1, _D), Wi0, bi0.reshape(1, _D),
      Wu1, bu1.reshape(1, _D), Wi1, bi1.reshape(1, _D))


def kernel(adj_norm, image_item_embeds, text_item_embeds, image_user_embeds,
           text_user_embeds, is_test, user_pre, item_pre, user_emb, item_emb,
           Wu0, bu0, Wi0, bi0, Wu1, bu1, Wi1, bi1):
    u, i = _run(user_pre, user_emb, image_user_embeds, text_user_embeds,
                item_pre, item_emb, image_item_embeds, text_item_embeds,
                Wu0, bu0, Wi0, bi0, Wu1, bu1, Wi1, bi1)
    return (u, i)
